# baseline (device time: 65452 ns/iter reference)
import jax
import jax.numpy as jnp
from jax import lax
from jax.experimental import pallas as pl
from jax.experimental.pallas import tpu as pltpu

N_DEV = 16
SQ = 1024
D_MODEL = 1024
HQ_PER = 8
DH = 128
HEAD_COLS = HQ_PER * DH
CHUNK = SQ // N_DEV
SCALE = 0.08838834764831843
BLK = 64

SLAB_MASKS = ((1, 3, 4, 8), (3, 1, 8, 4), (4, 8, 1, 3))
SLAB_COLS = ((0, 384), (384, 768), (768, 1024))
N_SLAB = len(SLAB_MASKS)
RS_SIZES = (512, 256, 128, 64)
RS_OFF = (0, 512, 768, 896)


def _bit(m, i):
    return (m >> i) & 1


_FUNC = {
    1: lambda c: _bit(c, 0) ^ _bit(c, 1),
    3: lambda c: _bit(c, 1),
    4: lambda c: _bit(c, 2),
    8: lambda c: _bit(c, 3),
}


def _pos(c, masks):
    return (8 * _FUNC[masks[0]](c) + 4 * _FUNC[masks[1]](c)
            + 2 * _FUNC[masks[2]](c) + _FUNC[masks[3]](c))


POS = tuple(tuple(_pos(c, mk) for c in range(N_DEV)) for mk in SLAB_MASKS)
INV = tuple(
    tuple({_pos(c, mk): c for c in range(N_DEV)}[q] for q in range(N_DEV))
    for mk in SLAB_MASKS)


def _body(x_ref, wq_ref, k_ref, v_ref, wo_ref, out_ref,
          q_ref, ctx_ref, acc_ref, send_ref, rs_recv_ref, ag_ref,
          bias_ref, send_sems, rs_sems, ag_sems):
    my = lax.axis_index("i")

    barrier_sem = pltpu.get_barrier_semaphore()
    for msk in (1, 3, 4, 8):
        pl.semaphore_signal(barrier_sem, inc=1, device_id=(my ^ msk,),
                            device_id_type=pl.DeviceIdType.MESH)
    pl.semaphore_wait(barrier_sem, 4)

    q_ref[...] = (jnp.dot(x_ref[...], wq_ref[...],
                          preferred_element_type=jnp.float32)
                  * SCALE).astype(jnp.bfloat16)

    qb = lax.broadcasted_iota(jnp.int32, (SQ, SQ), 0) // BLK
    kb = lax.broadcasted_iota(jnp.int32, (SQ, SQ), 1) // BLK
    bias_ref[...] = jnp.where(kb <= qb, 0.0, -30000.0).astype(jnp.float32)

    HALF = SQ // 2
    for h in range(HQ_PER):
        sl = slice(h * DH, (h + 1) * DH)
        for (r0, r1, ncols) in ((0, HALF, HALF), (HALF, SQ, SQ)):
            qh = q_ref[r0:r1, sl]
            kh = k_ref[0:ncols, sl]
            s = lax.dot_general(qh, kh, (((1,), (1,)), ((), ())),
                                preferred_element_type=jnp.float32)
            w = jnp.exp(s + bias_ref[r0:r1, 0:ncols])
            recip = 1.0 / jnp.sum(w, axis=1, keepdims=True)
            u = jnp.dot(w.astype(jnp.bfloat16), v_ref[0:ncols, sl],
                        preferred_element_type=jnp.float32)
            ctx_ref[r0:r1, sl] = (u * recip).astype(jnp.bfloat16)

    b0 = my & 1
    b1 = (my >> 1) & 1
    func = {1: b0 ^ b1, 3: b1, 4: (my >> 2) & 1, 8: (my >> 3) & 1}

    lo = [my * 0 for _ in range(N_SLAB)]
    rdmas_k0 = []
    keep_los_k0 = []
    for i in range(N_SLAB):
        c0, c1 = SLAB_COLS[i]
        val_i = jnp.dot(ctx_ref[...], wo_ref[:, c0:c1],
                        preferred_element_type=jnp.float32)
        for q in range(N_DEV):
            cq = INV[i][q]
            acc_ref[q * CHUNK:(q + 1) * CHUNK, c0:c1] = (
                val_i[cq * CHUNK:(cq + 1) * CHUNK, :])
        s = RS_SIZES[0]
        beta = func[SLAB_MASKS[i][0]]
        send_lo = pl.multiple_of(lo[i] + (1 - beta) * s, 64)
        keep_los_k0.append(pl.multiple_of(lo[i] + beta * s, 64))
        send_ref[0:s, c0:c1] = (
            acc_ref[pl.ds(send_lo, s), c0:c1].astype(jnp.bfloat16))
        rdma = pltpu.make_async_remote_copy(
            src_ref=send_ref.at[0:s, c0:c1],
            dst_ref=rs_recv_ref.at[RS_OFF[0]:RS_OFF[0] + s, c0:c1],
            send_sem=send_sems.at[i],
            recv_sem=rs_sems.at[i, 0],
            device_id=(my ^ SLAB_MASKS[i][0],),
            device_id_type=pl.DeviceIdType.MESH,
        )
        rdma.start()
        rdmas_k0.append(rdma)

    for k in range(3):
        s = RS_SIZES[k]
        if k == 0:
            rdmas = rdmas_k0
            keep_los = keep_los_k0
        else:
            rdmas = []
            keep_los = []
            for i in range(N_SLAB):
                c0, c1 = SLAB_COLS[i]
                beta = func[SLAB_MASKS[i][k]]
                send_lo = pl.multiple_of(lo[i] + (1 - beta) * s, 64)
                keep_los.append(pl.multiple_of(lo[i] + beta * s, 64))
                send_ref[0:s, c0:c1] = (
                    acc_ref[pl.ds(send_lo, s), c0:c1].astype(jnp.bfloat16))
                rdma = pltpu.make_async_remote_copy(
                    src_ref=send_ref.at[0:s, c0:c1],
                    dst_ref=rs_recv_ref.at[RS_OFF[k]:RS_OFF[k] + s, c0:c1],
                    send_sem=send_sems.at[i],
                    recv_sem=rs_sems.at[i, k],
                    device_id=(my ^ SLAB_MASKS[i][k],),
                    device_id_type=pl.DeviceIdType.MESH,
                )
                rdma.start()
                rdmas.append(rdma)
        for i in range(N_SLAB):
            c0, c1 = SLAB_COLS[i]
            rdmas[i].wait()
            acc_ref[pl.ds(keep_los[i], s), c0:c1] = (
                acc_ref[pl.ds(keep_los[i], s), c0:c1]
                + rs_recv_ref[RS_OFF[k]:RS_OFF[k] + s, c0:c1].astype(
                    jnp.float32))
            lo[i] = keep_los[i]

    rdmas = []
    sls = []
    for i in range(N_SLAB):
        c0, c1 = SLAB_COLS[i]
        sl_i = pl.ds(pl.multiple_of(lo[i], 128), 128)
        sls.append(sl_i)
        send_ref[0:128, c0:c1] = acc_ref[sl_i, c0:c1].astype(jnp.bfloat16)
        rdma = pltpu.make_async_remote_copy(
            src_ref=send_ref.at[0:128, c0:c1],
            dst_ref=rs_recv_ref.at[896:1024, c0:c1],
            send_sem=send_sems.at[i],
            recv_sem=rs_sems.at[i, 3],
            device_id=(my ^ SLAB_MASKS[i][3],),
            device_id_type=pl.DeviceIdType.MESH,
        )
        rdma.start()
        rdmas.append(rdma)
    for i in range(N_SLAB):
        c0, c1 = SLAB_COLS[i]
        rdmas[i].wait()
        ag_ref[sls[i], c0:c1] = (
            acc_ref[sls[i], c0:c1]
            + rs_recv_ref[896:1024, c0:c1].astype(jnp.float32)
        ).astype(jnp.bfloat16)

    for k in reversed(range(3)):
        sz = 128 << (2 - k)
        rdmas = []
        for i in range(N_SLAB):
            c0, c1 = SLAB_COLS[i]
            sl_i = pl.ds(pl.multiple_of(lo[i], 64), sz)
            rdma = pltpu.make_async_remote_copy(
                src_ref=ag_ref.at[sl_i, c0:c1],
                dst_ref=ag_ref.at[sl_i, c0:c1],
                send_sem=send_sems.at[i],
                recv_sem=ag_sems.at[i, k],
                device_id=(my ^ SLAB_MASKS[i][k],),
                device_id_type=pl.DeviceIdType.MESH,
            )
            rdma.start()
            rdmas.append(rdma)
        for i in range(N_SLAB):
            rdmas[i].wait()
            lo[i] = lo[i] - (lo[i] & sz)

    for b in range(N_DEV):
        for i in range(N_SLAB):
            c0, c1 = SLAB_COLS[i]
            pb = POS[i][b]
            out_ref[b * CHUNK:(b + 1) * CHUNK, c0:c1] = (
                ag_ref[pb * CHUNK:(pb + 1) * CHUNK, c0:c1].astype(
                    jnp.float32))


def kernel(x, Wq, K_ext, V_ext, Wo):
    my = lax.axis_index("i")
    x2 = x.reshape(SQ, D_MODEL).astype(jnp.bfloat16)
    k2 = K_ext.reshape(SQ, HEAD_COLS).astype(jnp.bfloat16)
    v2 = V_ext.reshape(SQ, HEAD_COLS).astype(jnp.bfloat16)
    wq_s = lax.dynamic_slice(
        Wq, (0, my * HEAD_COLS), (D_MODEL, HEAD_COLS)).astype(jnp.bfloat16)
    wo_s = lax.dynamic_slice(
        Wo, (my * HEAD_COLS, 0), (HEAD_COLS, D_MODEL)).astype(jnp.bfloat16)

    out = pl.pallas_call(
        _body,
        out_shape=jax.ShapeDtypeStruct((SQ, D_MODEL), jnp.float32),
        in_specs=[pl.BlockSpec(memory_space=pltpu.VMEM)] * 5,
        out_specs=pl.BlockSpec(memory_space=pltpu.VMEM),
        scratch_shapes=[
            pltpu.VMEM((SQ, HEAD_COLS), jnp.bfloat16),
            pltpu.VMEM((SQ, HEAD_COLS), jnp.bfloat16),
            pltpu.VMEM((SQ, D_MODEL), jnp.float32),
            pltpu.VMEM((512, D_MODEL), jnp.bfloat16),
            pltpu.VMEM((1024, D_MODEL), jnp.bfloat16),
            pltpu.VMEM((SQ, D_MODEL), jnp.bfloat16),
            pltpu.VMEM((SQ, SQ), jnp.float32),
            pltpu.SemaphoreType.DMA((N_SLAB,)),
            pltpu.SemaphoreType.DMA((N_SLAB, 4)),
            pltpu.SemaphoreType.DMA((N_SLAB, 4)),
        ],
        compiler_params=pltpu.CompilerParams(collective_id=0),
    )(x2, wq_s, k2, v2, wo_s)
    return out.reshape(1, SQ, D_MODEL)


# device time: 65226 ns/iter; 1.0035x vs baseline; 1.0035x over previous
import jax
import jax.numpy as jnp
from jax import lax
from jax.experimental import pallas as pl
from jax.experimental.pallas import tpu as pltpu

N_DEV = 16
SQ = 1024
D_MODEL = 1024
HQ_PER = 8
DH = 128
HEAD_COLS = HQ_PER * DH
CHUNK = SQ // N_DEV
SCALE = 0.08838834764831843
BLK = 64

SLAB_MASKS = ((1, 3, 4, 8), (3, 1, 8, 4), (4, 8, 1, 3))
SLAB_COLS = ((0, 384), (384, 768), (768, 1024))
N_SLAB = len(SLAB_MASKS)
RS_SIZES = (512, 256, 128, 64)
RS_OFF = (0, 512, 768, 896)


def _bit(m, i):
    return (m >> i) & 1


_FUNC = {
    1: lambda c: _bit(c, 0) ^ _bit(c, 1),
    3: lambda c: _bit(c, 1),
    4: lambda c: _bit(c, 2),
    8: lambda c: _bit(c, 3),
}


def _pos(c, masks):
    return (8 * _FUNC[masks[0]](c) + 4 * _FUNC[masks[1]](c)
            + 2 * _FUNC[masks[2]](c) + _FUNC[masks[3]](c))


POS = tuple(tuple(_pos(c, mk) for c in range(N_DEV)) for mk in SLAB_MASKS)
INV = tuple(
    tuple({_pos(c, mk): c for c in range(N_DEV)}[q] for q in range(N_DEV))
    for mk in SLAB_MASKS)


def _body(x_ref, wq_ref, k_ref, v_ref, wo_ref, out_ref,
          q_ref, ctx_ref, acc_ref, send_ref, rs_recv_ref, ag_ref,
          bias_ref, send_sems, rs_sems, ag_sems):
    my = lax.axis_index("i")

    barrier_sem = pltpu.get_barrier_semaphore()
    for msk in (1, 3, 4, 8):
        pl.semaphore_signal(barrier_sem, inc=1, device_id=(my ^ msk,),
                            device_id_type=pl.DeviceIdType.MESH)
    pl.semaphore_wait(barrier_sem, 4)

    q_ref[...] = jnp.dot(x_ref[...], wq_ref[...],
                         preferred_element_type=jnp.float32).astype(
        jnp.bfloat16)

    qb = lax.broadcasted_iota(jnp.int32, (SQ, SQ), 0) // BLK
    kb = lax.broadcasted_iota(jnp.int32, (SQ, SQ), 1) // BLK
    bias_ref[...] = jnp.where(kb <= qb, 0.0, -30000.0).astype(jnp.float32)

    HALF = SQ // 2
    for h in range(HQ_PER):
        sl = slice(h * DH, (h + 1) * DH)
        for (r0, r1, ncols) in ((0, HALF, HALF), (HALF, SQ, SQ)):
            qh = q_ref[r0:r1, sl]
            kh = k_ref[0:ncols, sl]
            s = lax.dot_general(qh, kh, (((1,), (1,)), ((), ())),
                                preferred_element_type=jnp.float32) * SCALE
            w = jnp.exp(s + bias_ref[r0:r1, 0:ncols])
            recip = 1.0 / jnp.sum(w, axis=1, keepdims=True)
            u = jnp.dot(w.astype(jnp.bfloat16), v_ref[0:ncols, sl],
                        preferred_element_type=jnp.float32)
            ctx_ref[r0:r1, sl] = (u * recip).astype(jnp.bfloat16)

    val = jnp.dot(ctx_ref[...], wo_ref[...],
                  preferred_element_type=jnp.float32)
    for q in range(N_DEV):
        for i in range(N_SLAB):
            c0, c1 = SLAB_COLS[i]
            cq = INV[i][q]
            acc_ref[q * CHUNK:(q + 1) * CHUNK, c0:c1] = (
                val[cq * CHUNK:(cq + 1) * CHUNK, c0:c1])

    b0 = my & 1
    b1 = (my >> 1) & 1
    func = {1: b0 ^ b1, 3: b1, 4: (my >> 2) & 1, 8: (my >> 3) & 1}

    lo = [my * 0 for _ in range(N_SLAB)]
    for k in range(3):
        s = RS_SIZES[k]
        rdmas = []
        keep_los = []
        for i in range(N_SLAB):
            c0, c1 = SLAB_COLS[i]
            beta = func[SLAB_MASKS[i][k]]
            send_lo = pl.multiple_of(lo[i] + (1 - beta) * s, 64)
            keep_lo = pl.multiple_of(lo[i] + beta * s, 64)
            keep_los.append(keep_lo)
            send_ref[0:s, c0:c1] = (
                acc_ref[pl.ds(send_lo, s), c0:c1].astype(jnp.bfloat16))
            rdma = pltpu.make_async_remote_copy(
                src_ref=send_ref.at[0:s, c0:c1],
                dst_ref=rs_recv_ref.at[RS_OFF[k]:RS_OFF[k] + s, c0:c1],
                send_sem=send_sems.at[i],
                recv_sem=rs_sems.at[i, k],
                device_id=(my ^ SLAB_MASKS[i][k],),
                device_id_type=pl.DeviceIdType.MESH,
            )
            rdma.start()
            rdmas.append(rdma)
        for i in range(N_SLAB):
            c0, c1 = SLAB_COLS[i]
            rdmas[i].wait()
            acc_ref[pl.ds(keep_los[i], s), c0:c1] = (
                acc_ref[pl.ds(keep_los[i], s), c0:c1]
                + rs_recv_ref[RS_OFF[k]:RS_OFF[k] + s, c0:c1].astype(
                    jnp.float32))
            lo[i] = keep_los[i]

    rdmas = []
    sls = []
    for i in range(N_SLAB):
        c0, c1 = SLAB_COLS[i]
        sl_i = pl.ds(pl.multiple_of(lo[i], 128), 128)
        sls.append(sl_i)
        send_ref[0:128, c0:c1] = acc_ref[sl_i, c0:c1].astype(jnp.bfloat16)
        rdma = pltpu.make_async_remote_copy(
            src_ref=send_ref.at[0:128, c0:c1],
            dst_ref=rs_recv_ref.at[896:1024, c0:c1],
            send_sem=send_sems.at[i],
            recv_sem=rs_sems.at[i, 3],
            device_id=(my ^ SLAB_MASKS[i][3],),
            device_id_type=pl.DeviceIdType.MESH,
        )
        rdma.start()
        rdmas.append(rdma)
    for i in range(N_SLAB):
        c0, c1 = SLAB_COLS[i]
        rdmas[i].wait()
        ag_ref[sls[i], c0:c1] = (
            acc_ref[sls[i], c0:c1]
            + rs_recv_ref[896:1024, c0:c1].astype(jnp.float32)
        ).astype(jnp.bfloat16)

    for k in reversed(range(3)):
        sz = 128 << (2 - k)
        rdmas = []
        for i in range(N_SLAB):
            c0, c1 = SLAB_COLS[i]
            sl_i = pl.ds(pl.multiple_of(lo[i], 64), sz)
            rdma = pltpu.make_async_remote_copy(
                src_ref=ag_ref.at[sl_i, c0:c1],
                dst_ref=ag_ref.at[sl_i, c0:c1],
                send_sem=send_sems.at[i],
                recv_sem=ag_sems.at[i, k],
                device_id=(my ^ SLAB_MASKS[i][k],),
                device_id_type=pl.DeviceIdType.MESH,
            )
            rdma.start()
            rdmas.append(rdma)
        for i in range(N_SLAB):
            rdmas[i].wait()
            lo[i] = lo[i] - (lo[i] & sz)

    for b in range(N_DEV):
        for i in range(N_SLAB):
            c0, c1 = SLAB_COLS[i]
            pb = POS[i][b]
            out_ref[b * CHUNK:(b + 1) * CHUNK, c0:c1] = (
                ag_ref[pb * CHUNK:(pb + 1) * CHUNK, c0:c1].astype(
                    jnp.float32))


def kernel(x, Wq, K_ext, V_ext, Wo):
    my = lax.axis_index("i")
    x2 = x.reshape(SQ, D_MODEL).astype(jnp.bfloat16)
    k2 = K_ext.reshape(SQ, HEAD_COLS).astype(jnp.bfloat16)
    v2 = V_ext.reshape(SQ, HEAD_COLS).astype(jnp.bfloat16)
    wq_s = lax.dynamic_slice(
        Wq, (0, my * HEAD_COLS), (D_MODEL, HEAD_COLS)).astype(jnp.bfloat16)
    wo_s = lax.dynamic_slice(
        Wo, (my * HEAD_COLS, 0), (HEAD_COLS, D_MODEL)).astype(jnp.bfloat16)

    out = pl.pallas_call(
        _body,
        out_shape=jax.ShapeDtypeStruct((SQ, D_MODEL), jnp.float32),
        in_specs=[pl.BlockSpec(memory_space=pltpu.VMEM)] * 5,
        out_specs=pl.BlockSpec(memory_space=pltpu.VMEM),
        scratch_shapes=[
            pltpu.VMEM((SQ, HEAD_COLS), jnp.bfloat16),
            pltpu.VMEM((SQ, HEAD_COLS), jnp.bfloat16),
            pltpu.VMEM((SQ, D_MODEL), jnp.float32),
            pltpu.VMEM((512, D_MODEL), jnp.bfloat16),
            pltpu.VMEM((1024, D_MODEL), jnp.bfloat16),
            pltpu.VMEM((SQ, D_MODEL), jnp.bfloat16),
            pltpu.VMEM((SQ, SQ), jnp.float32),
            pltpu.SemaphoreType.DMA((N_SLAB,)),
            pltpu.SemaphoreType.DMA((N_SLAB, 4)),
            pltpu.SemaphoreType.DMA((N_SLAB, 4)),
        ],
        compiler_params=pltpu.CompilerParams(collective_id=0),
    )(x2, wq_s, k2, v2, wo_s)
    return out.reshape(1, SQ, D_MODEL)
